# baseline (speedup 1.0000x reference)
import functools
import jax, jax.numpy as jnp
from jax import lax
from jax.experimental import pallas as pl
from jax.experimental.pallas import tpu as pltpu
from jax.experimental.pallas import tpu_sc as plsc

_V = 1000
_mesh = plsc.VectorSubcoreMesh(core_axis_name="c", subcore_axis_name="s",
                               num_cores=1, num_subcores=1)

@functools.partial(
    pl.kernel, mesh=_mesh,
    out_type=jax.ShapeDtypeStruct((16,), jnp.float32),
    scratch_types=[pltpu.VMEM((16,), jnp.float32)],
    compiler_params=pltpu.CompilerParams(needs_layout_passes=False),
)
def _probe(tok_hbm, out_hbm, buf_v):
    buf_v[...] = jnp.zeros((16,), jnp.float32)
    pltpu.sync_copy(buf_v, out_hbm)

def kernel(input):
    return jnp.broadcast_to(_probe(input.reshape(2))[:1], (1, _V))


# speedup vs baseline: 1.0237x; 1.0237x over previous
import functools

import jax
import jax.numpy as jnp
from jax import lax
from jax.experimental import pallas as pl
from jax.experimental.pallas import tpu as pltpu
from jax.experimental.pallas import tpu_sc as plsc

_V = 1000
_CHUNK = 128
_L = 16
_NW = 8
_TAIL = _V - (_NW - 1) * _CHUNK

_mesh = plsc.VectorSubcoreMesh(core_axis_name="c", subcore_axis_name="s",
                               num_cores=1, num_subcores=8)


@functools.partial(
    pl.kernel,
    mesh=_mesh,
    out_type=jax.ShapeDtypeStruct((_V,), jnp.float32),
    scratch_types=[
        pltpu.VMEM((_L,), jnp.int32),
        pltpu.VMEM((_CHUNK,), jnp.float32),
    ],
    compiler_params=pltpu.CompilerParams(needs_layout_passes=False),
)
def _bow_sc(tok_hbm, out_hbm, tok_v, buf_v):
    wid = lax.axis_index("s")
    base = wid * _CHUNK

    pltpu.sync_copy(tok_hbm, tok_v.at[pl.ds(0, 2)])
    toks = tok_v[...]

    zeros = jnp.zeros((_L,), jnp.float32)
    for i in range(_CHUNK // _L):
        buf_v[pl.ds(i * _L, _L)] = zeros

    lane = lax.iota(jnp.int32, _L)
    loc = toks - base
    in_range = (loc >= 0) & (loc < _CHUNK)
    loc_c = jnp.clip(loc, 0, _CHUNK - 1)
    ones = jnp.ones((_L,), jnp.float32)

    plsc.addupdate_scatter(buf_v, [loc_c], ones, mask=(lane == 0) & in_range)
    plsc.addupdate_scatter(buf_v, [loc_c], ones, mask=(lane == 1) & in_range)

    @pl.when(wid < _NW - 1)
    def _():
        pltpu.sync_copy(buf_v, out_hbm.at[pl.ds(base, _CHUNK)])

    @pl.when(wid == _NW - 1)
    def _():
        pltpu.sync_copy(buf_v.at[pl.ds(0, _TAIL)], out_hbm.at[pl.ds(base, _TAIL)])


def kernel(input):
    return _bow_sc(input.reshape(2)).reshape(1, _V)


# speedup vs baseline: 1.0249x; 1.0012x over previous
import functools

import jax
import jax.numpy as jnp
from jax import lax
from jax.experimental import pallas as pl
from jax.experimental.pallas import tpu as pltpu
from jax.experimental.pallas import tpu_sc as plsc

_V = 1000
_CHUNK = 128
_L = 16
_NW = 8
_TAIL = _V - (_NW - 1) * _CHUNK

_mesh = plsc.VectorSubcoreMesh(core_axis_name="c", subcore_axis_name="s",
                               num_cores=1, num_subcores=8)


@functools.partial(
    pl.kernel,
    mesh=_mesh,
    out_type=jax.ShapeDtypeStruct((_V,), jnp.float32),
    scratch_types=[
        pltpu.VMEM((_L,), jnp.int32),
        pltpu.VMEM((_CHUNK,), jnp.float32),
    ],
    compiler_params=pltpu.CompilerParams(needs_layout_passes=False, skip_device_barrier=True, disable_semaphore_checks=True, disable_bounds_checks=True),
)
def _bow_sc(tok_hbm, out_hbm, tok_v, buf_v):
    wid = lax.axis_index("s")
    base = wid * _CHUNK

    pltpu.sync_copy(tok_hbm, tok_v.at[pl.ds(0, 2)])
    toks = tok_v[...]

    zeros = jnp.zeros((_L,), jnp.float32)
    for i in range(_CHUNK // _L):
        buf_v[pl.ds(i * _L, _L)] = zeros

    lane = lax.iota(jnp.int32, _L)
    loc = toks - base
    in_range = (loc >= 0) & (loc < _CHUNK)
    loc_c = jnp.clip(loc, 0, _CHUNK - 1)
    ones = jnp.ones((_L,), jnp.float32)

    plsc.addupdate_scatter(buf_v, [loc_c], ones, mask=(lane == 0) & in_range)
    plsc.addupdate_scatter(buf_v, [loc_c], ones, mask=(lane == 1) & in_range)

    @pl.when(wid < _NW - 1)
    def _():
        pltpu.sync_copy(buf_v, out_hbm.at[pl.ds(base, _CHUNK)])

    @pl.when(wid == _NW - 1)
    def _():
        pltpu.sync_copy(buf_v.at[pl.ds(0, _TAIL)], out_hbm.at[pl.ds(base, _TAIL)])


def kernel(input):
    return _bow_sc(input.reshape(2)).reshape(1, _V)
